# baseline (speedup 1.0000x reference)
import jax
import jax.numpy as jnp
from jax import lax
from jax.experimental import pallas as pl
from jax.experimental.pallas import tpu as pltpu
from jax.experimental.pallas import tpu_sc as plsc

BATCH = 16384
HIST_LEN = 50
D = 64
V = 1000000
N = BATCH * HIST_LEN
NC, NS = 2, 16
NW = NC * NS
C = 128
CPW = N // (NW * C)
NBUF = 8
NGRP = CPW // NBUF
NWBUF = 2
BB = BATCH // C
TP = D * 2 + 1

BLK = 128
RPER = 246


def _repack_body(tbl_hbm, out_hbm, bin_, bout, gsems, wsems):
    wid = lax.axis_index("s") * NC + lax.axis_index("c")
    i0 = wid * RPER

    def fire(i, b):
        r0 = pl.multiple_of(jnp.minimum(i * BLK, V - BLK), 64)
        pltpu.async_copy(tbl_hbm.at[pl.ds(r0, BLK), :], bin_[b], gsems[b])

    def wait_in(b):
        pltpu.make_async_copy(tbl_hbm.at[pl.ds(0, BLK), :], bin_[b],
                              gsems[b]).wait()

    def repack(b, t):
        @plsc.parallel_loop(0, BLK // 2, 1, unroll=4)
        def body(p):
            for par in range(2):
                for k in range(D // 16):
                    g = bin_[b][2 * p + par, pl.ds(16 * k, 16)]
                    bout[t][p, pl.ds(par * D + 16 * k, 16)] = g

    def fire_out(i, t):
        r0 = pl.multiple_of(jnp.minimum(i * BLK, V - BLK), 64)
        p0 = pl.multiple_of(r0 // 2, 32)
        pltpu.async_copy(bout[t], out_hbm.at[pl.ds(p0, BLK // 2), :],
                         wsems[t])

    def wait_out(t):
        pltpu.make_async_copy(bout[t], out_hbm.at[pl.ds(0, BLK // 2), :],
                              wsems[t]).wait()

    fire(i0, 0)

    def pair(g, carry):
        i = i0 + g * 2
        fire(i + 1, 1)
        wait_in(0)
        repack(0, 0)
        fire_out(i, 0)
        fire(i + 2, 0)
        wait_in(1)
        repack(1, 1)
        fire_out(i + 1, 1)
        wait_out(0)
        wait_out(1)
        return carry

    lax.fori_loop(0, RPER // 2, pair, 0)
    wait_in(0)


def _gather_body(idx_hbm, table_hbm, out_hbm, idx_v, rows, trows, gsem, wsem):
    wid = lax.axis_index("s") * NC + lax.axis_index("c")
    q0 = wid * CPW
    pltpu.sync_copy(idx_hbm.at[pl.ds(q0, CPW)], idx_v)

    lane = lax.iota(jnp.int32, 16)
    drow = [lane + 16 * k for k in range(D // 16)]

    def fire_gather(j, b):
        pltpu.async_copy(table_hbm.at[idx_v.at[j]], rows[b], gsem[b])

    def wait_gather(b):
        pltpu.make_async_copy(table_hbm.at[idx_v.at[0]], rows[b], gsem[b]).wait()

    def fire_write(j, t):
        q = q0 + j
        for i in range(D // 8):
            pltpu.async_copy(trows[t].at[pl.ds(8 * i, 8), :C],
                             out_hbm.at[q // BB, i, q % BB], wsem[t])

    def wait_write(t):
        for i in range(D // 8):
            pltpu.make_async_copy(trows[t].at[pl.ds(8 * i, 8), :C],
                                  out_hbm.at[0, i, 0], wsem[t]).wait()

    def transpose_block(b, t):
        @plsc.parallel_loop(0, C, 1, unroll=4, carry=lane * 0)
        def body(l, lsplat):
            for k in range(D // 16):
                g = rows[b][l, pl.ds(16 * k, 16)]
                plsc.store_scatter(trows[t], [drow[k], lsplat], g)
            return lsplat + 1

    def step(j, b, do_wait_write, do_fire):
        t = b % NWBUF
        wait_gather(b)
        if do_wait_write:
            wait_write(t)
        transpose_block(b, t)
        fire_write(j, t)
        if do_fire:
            fire_gather(j + NBUF - 1, (b + NBUF - 1) % NBUF)

    for b in range(NBUF - 1):
        fire_gather(b, b)
    for b in range(NBUF):
        step(b, b, do_wait_write=(b >= NWBUF), do_fire=True)

    def group(g, carry):
        for b in range(NBUF):
            step(g * NBUF + b, b, do_wait_write=True, do_fire=True)
        return carry

    lax.fori_loop(1, NGRP - 1, group, 0)

    j0 = (NGRP - 1) * NBUF
    for b in range(NBUF):
        step(j0 + b, b, do_wait_write=True, do_fire=(b == 0))

    for t in range(NWBUF):
        wait_write(t)


@jax.jit
def kernel(traj, emb_weight):
    idx = traj.astype(jnp.int32).T.reshape(N // C, C)
    tbl2 = pl.kernel(
        _repack_body,
        out_type=jax.ShapeDtypeStruct((V // 2, 2 * D), jnp.float32),
        mesh=plsc.VectorSubcoreMesh(core_axis_name="c", subcore_axis_name="s"),
        compiler_params=pltpu.CompilerParams(
            use_tc_tiling_on_sc=True,
            needs_layout_passes=False,
            disable_bounds_checks=True,
        ),
        scratch_types=[
            [pltpu.VMEM((BLK, D), jnp.float32) for _ in range(2)],
            [pltpu.VMEM((BLK // 2, 2 * D), jnp.float32) for _ in range(2)],
            [pltpu.SemaphoreType.DMA for _ in range(2)],
            [pltpu.SemaphoreType.DMA for _ in range(2)],
        ],
    )(emb_weight)
    tbl = tbl2.reshape(V, D)
    out = pl.kernel(
        _gather_body,
        out_type=jax.ShapeDtypeStruct((HIST_LEN, D // 8, BB, 8, C), jnp.float32),
        mesh=plsc.VectorSubcoreMesh(core_axis_name="c", subcore_axis_name="s"),
        compiler_params=pltpu.CompilerParams(
            use_tc_tiling_on_sc=False,
            needs_layout_passes=False,
            disable_bounds_checks=True,
        ),
        scratch_types=[
            pltpu.VMEM((CPW, C), jnp.int32),
            [pltpu.VMEM((C, D), jnp.float32) for _ in range(NBUF)],
            [pltpu.VMEM((D, TP), jnp.float32) for _ in range(NWBUF)],
            [pltpu.SemaphoreType.DMA for _ in range(NBUF)],
            [pltpu.SemaphoreType.DMA for _ in range(NWBUF)],
        ],
    )(idx, tbl)
    out = out.transpose(0, 1, 3, 2, 4).reshape(HIST_LEN, D, BATCH)
    return out.transpose(2, 0, 1)


# speedup vs baseline: 1.0750x; 1.0750x over previous
import jax
import jax.numpy as jnp
from jax import lax
from jax.experimental import pallas as pl
from jax.experimental.pallas import tpu as pltpu
from jax.experimental.pallas import tpu_sc as plsc

BATCH = 16384
HIST_LEN = 50
D = 64
N = BATCH * HIST_LEN
NC, NS = 2, 16
NW = NC * NS
C = 128
CPW = N // (NW * C)
NBUF = 8
NGRP = CPW // NBUF
NWBUF = 2
BB = BATCH // C
TP = D * 2 + 1


def _emb_body(idx_hbm, table_hbm, out_hbm, idx_v, rows, trows, gsem, wsem):
    wid = lax.axis_index("s") * NC + lax.axis_index("c")
    q0 = wid * CPW
    pltpu.sync_copy(idx_hbm.at[pl.ds(q0, CPW)], idx_v)

    lane = lax.iota(jnp.int32, 16)
    drow = [lane + 16 * k for k in range(D // 16)]

    def fire_gather(j, b):
        pltpu.async_copy(table_hbm.at[idx_v.at[j]], rows[b], gsem[b])

    def wait_gather(b):
        pltpu.make_async_copy(table_hbm.at[idx_v.at[0]], rows[b], gsem[b]).wait()

    def fire_write(j, t):
        q = q0 + j
        for i in range(D // 8):
            pltpu.async_copy(trows[t].at[pl.ds(8 * i, 8), :C],
                             out_hbm.at[q // BB, i, q % BB], wsem[t])

    def wait_write(t):
        for i in range(D // 8):
            pltpu.make_async_copy(trows[t].at[pl.ds(8 * i, 8), :C],
                                  out_hbm.at[0, i, 0], wsem[t]).wait()

    def transpose_block(b, t):
        @plsc.parallel_loop(0, C, 1, unroll=4, carry=lane * 0)
        def body(l, lsplat):
            for k in range(D // 16):
                g = rows[b][l, pl.ds(16 * k, 16)]
                plsc.store_scatter(trows[t], [drow[k], lsplat], g)
            return lsplat + 1

    def step(j, b, do_wait_write, do_fire):
        t = b % NWBUF
        wait_gather(b)
        if do_wait_write:
            wait_write(t)
        transpose_block(b, t)
        fire_write(j, t)
        if do_fire:
            fire_gather(j + NBUF - 1, (b + NBUF - 1) % NBUF)

    for b in range(NBUF - 1):
        fire_gather(b, b)
    for b in range(NBUF):
        step(b, b, do_wait_write=(b >= NWBUF), do_fire=True)

    def group(g, carry):
        for b in range(NBUF):
            step(g * NBUF + b, b, do_wait_write=True, do_fire=True)
        return carry

    lax.fori_loop(1, NGRP - 1, group, 0)

    j0 = (NGRP - 1) * NBUF
    for b in range(NBUF):
        step(j0 + b, b, do_wait_write=True, do_fire=(b == 0))

    for t in range(NWBUF):
        wait_write(t)


@jax.jit
def kernel(traj, emb_weight):
    idx = traj.astype(jnp.int32).T.reshape(N // C, C)
    out = pl.kernel(
        _emb_body,
        out_type=jax.ShapeDtypeStruct((HIST_LEN, D // 8, BB, 8, C), jnp.float32),
        mesh=plsc.VectorSubcoreMesh(core_axis_name="c", subcore_axis_name="s"),
        compiler_params=pltpu.CompilerParams(
            use_tc_tiling_on_sc=False,
            needs_layout_passes=False,
            disable_bounds_checks=True,
        ),
        scratch_types=[
            pltpu.VMEM((CPW, C), jnp.int32),
            [pltpu.VMEM((C, D), jnp.float32) for _ in range(NBUF)],
            [pltpu.VMEM((D, TP), jnp.float32) for _ in range(NWBUF)],
            [pltpu.SemaphoreType.DMA for _ in range(NBUF)],
            [pltpu.SemaphoreType.DMA for _ in range(NWBUF)],
        ],
    )(idx, emb_weight)
    out = out.transpose(0, 1, 3, 2, 4).reshape(HIST_LEN, D, BATCH)
    return out.transpose(2, 0, 1)
